# baseline (speedup 1.0000x reference)
import functools

import jax
import jax.numpy as jnp
from jax import lax
from jax.experimental import pallas as pl
from jax.experimental.pallas import tpu as pltpu
from jax.experimental.pallas import tpu_sc as plsc

N_IDS = 16384 * 6
D = 128

_info = plsc.get_sparse_core_info()
NC, NS = _info.num_cores, _info.num_subcores
NW = NC * NS
B_PER_W = N_IDS // NW
CHUNK = 128
N_CHUNKS = B_PER_W // CHUNK

_mesh = plsc.VectorSubcoreMesh(core_axis_name="c", subcore_axis_name="s")


@functools.partial(
    pl.kernel,
    mesh=_mesh,
    out_type=jax.ShapeDtypeStruct((N_IDS, D), jnp.float32),
    scratch_types=[
        pltpu.VMEM((B_PER_W,), jnp.int32),
        pltpu.VMEM((CHUNK, D), jnp.float32),
        pltpu.SemaphoreType.DMA,
    ],
)
def _embed(ids_hbm, table_hbm, out_hbm, idx_v, buf, gsem):
    wid = lax.axis_index("s") * NC + lax.axis_index("c")
    base = wid * B_PER_W
    pltpu.sync_copy(ids_hbm.at[pl.ds(base, B_PER_W)], idx_v)

    def step(i, carry):
        pltpu.async_copy(
            table_hbm.at[idx_v.at[pl.ds(i * CHUNK, CHUNK)]], buf, gsem
        ).wait()
        pltpu.sync_copy(buf, out_hbm.at[pl.ds(base + i * CHUNK, CHUNK)])
        return carry

    lax.fori_loop(0, N_CHUNKS, step, 0)


def kernel(camera_ids, camera_embeddings):
    ids = camera_ids.reshape(-1).astype(jnp.int32)
    out = _embed(ids, camera_embeddings)
    return out.reshape(camera_ids.shape[0], camera_ids.shape[1], D)


# speedup vs baseline: 1.0005x; 1.0005x over previous
import functools

import jax
import jax.numpy as jnp
from jax import lax
from jax.experimental import pallas as pl
from jax.experimental.pallas import tpu as pltpu
from jax.experimental.pallas import tpu_sc as plsc

N_IDS = 16384 * 6
D = 128

_info = plsc.get_sparse_core_info()
NC, NS = _info.num_cores, _info.num_subcores
NW = NC * NS
B_PER_W = N_IDS // NW
CHUNK = 128
N_CHUNKS = B_PER_W // CHUNK
NBUF = 4
LOOKAHEAD = 3

_mesh = plsc.VectorSubcoreMesh(core_axis_name="c", subcore_axis_name="s")


@functools.partial(
    pl.kernel,
    mesh=_mesh,
    out_type=jax.ShapeDtypeStruct((N_IDS, D), jnp.float32),
    scratch_types=[
        pltpu.VMEM((B_PER_W,), jnp.int32),
        pltpu.VMEM((NBUF, CHUNK, D), jnp.float32),
        pltpu.SemaphoreType.DMA,
        pltpu.SemaphoreType.DMA,
    ],
)
def _embed(ids_hbm, table_hbm, out_hbm, idx_v, bufs, gsem, ssem):
    wid = lax.axis_index("s") * NC + lax.axis_index("c")
    base = wid * B_PER_W
    pltpu.sync_copy(ids_hbm.at[pl.ds(base, B_PER_W)], idx_v)

    gathers = []
    stores = []

    def gstart(i):
        gathers.append(
            pltpu.async_copy(
                table_hbm.at[idx_v.at[pl.ds(i * CHUNK, CHUNK)]],
                bufs.at[i % NBUF],
                gsem,
            )
        )

    def sstart(i):
        stores.append(
            pltpu.async_copy(
                bufs.at[i % NBUF], out_hbm.at[pl.ds(base + i * CHUNK, CHUNK)], ssem
            )
        )

    for i in range(N_CHUNKS):
        if i >= NBUF:
            stores[i - NBUF].wait()
        gstart(i)
        if i >= LOOKAHEAD:
            j = i - LOOKAHEAD
            gathers[j].wait()
            sstart(j)
    for j in range(N_CHUNKS - LOOKAHEAD, N_CHUNKS):
        gathers[j].wait()
        sstart(j)
    for j in range(N_CHUNKS - NBUF, N_CHUNKS):
        stores[j].wait()


def kernel(camera_ids, camera_embeddings):
    ids = camera_ids.reshape(-1).astype(jnp.int32)
    out = _embed(ids, camera_embeddings)
    return out.reshape(camera_ids.shape[0], camera_ids.shape[1], D)


# speedup vs baseline: 4.4591x; 4.4569x over previous
import functools

import jax
import jax.numpy as jnp
from jax import lax
from jax.experimental import pallas as pl
from jax.experimental.pallas import tpu as pltpu
from jax.experimental.pallas import tpu_sc as plsc

N_IDS = 16384 * 6
D = 128

_info = plsc.get_sparse_core_info()
NC, NS = _info.num_cores, _info.num_subcores
NW = NC * NS
B_PER_W = N_IDS // NW
CHUNK = 128
N_CHUNKS = B_PER_W // CHUNK
NBUF = 4
LOOKAHEAD = 3

_mesh = plsc.VectorSubcoreMesh(core_axis_name="c", subcore_axis_name="s")


@functools.partial(
    pl.kernel,
    mesh=_mesh,
    out_type=jax.ShapeDtypeStruct((N_IDS, D), jnp.float32),
    scratch_types=[
        pltpu.VMEM((B_PER_W,), jnp.int32),
        pltpu.VMEM((NBUF, CHUNK, D), jnp.float32),
        pltpu.VMEM_SHARED((8, D), jnp.float32),
        pltpu.SemaphoreType.DMA,
        pltpu.SemaphoreType.DMA,
    ],
)
def _embed(ids_hbm, table_hbm, out_hbm, idx_v, bufs, table_sp, gsem, ssem):
    sid = lax.axis_index("s")
    wid = sid * NC + lax.axis_index("c")
    base = wid * B_PER_W

    @pl.when(sid == 0)
    def _stage_table():
        pltpu.sync_copy(table_hbm, table_sp)

    pltpu.sync_copy(ids_hbm.at[pl.ds(base, B_PER_W)], idx_v)
    plsc.subcore_barrier()

    gathers = []
    stores = []

    def gstart(i):
        gathers.append(
            pltpu.async_copy(
                table_sp.at[idx_v.at[pl.ds(i * CHUNK, CHUNK)]],
                bufs.at[i % NBUF],
                gsem,
            )
        )

    def sstart(i):
        stores.append(
            pltpu.async_copy(
                bufs.at[i % NBUF], out_hbm.at[pl.ds(base + i * CHUNK, CHUNK)], ssem
            )
        )

    for i in range(N_CHUNKS):
        if i >= NBUF:
            stores[i - NBUF].wait()
        gstart(i)
        if i >= LOOKAHEAD:
            j = i - LOOKAHEAD
            gathers[j].wait()
            sstart(j)
    for j in range(N_CHUNKS - LOOKAHEAD, N_CHUNKS):
        gathers[j].wait()
        sstart(j)
    for j in range(N_CHUNKS - NBUF, N_CHUNKS):
        stores[j].wait()


def kernel(camera_ids, camera_embeddings):
    ids = camera_ids.reshape(-1).astype(jnp.int32)
    out = _embed(ids, camera_embeddings)
    return out.reshape(camera_ids.shape[0], camera_ids.shape[1], D)


# speedup vs baseline: 7.7508x; 1.7382x over previous
import functools

import jax
import jax.numpy as jnp
from jax import lax
from jax.experimental import pallas as pl
from jax.experimental.pallas import tpu as pltpu
from jax.experimental.pallas import tpu_sc as plsc

N_ROWS = 16384
N_COLS = 6
N_IDS = N_ROWS * N_COLS
D = 128

_info = plsc.get_sparse_core_info()
NC, NS = _info.num_cores, _info.num_subcores
NW = NC * NS
B_PER_W = N_IDS // NW
R_PER_W = N_ROWS // NW
CR = 16
CHUNK = CR * N_COLS
N_CHUNKS = B_PER_W // CHUNK
NBUF = 4
LOOKAHEAD = 3

_mesh = plsc.VectorSubcoreMesh(core_axis_name="c", subcore_axis_name="s")


@functools.partial(
    pl.kernel,
    mesh=_mesh,
    out_type=jax.ShapeDtypeStruct((N_ROWS, N_COLS, D), jnp.float32),
    scratch_types=[
        pltpu.VMEM((B_PER_W,), jnp.int32),
        pltpu.VMEM((NBUF, CHUNK, D), jnp.float32),
        pltpu.VMEM_SHARED((8, D), jnp.float32),
        pltpu.SemaphoreType.DMA,
        pltpu.SemaphoreType.DMA,
    ],
)
def _embed(ids_hbm, table_hbm, out_hbm, idx_v, bufs, table_sp, gsem, ssem):
    sid = lax.axis_index("s")
    wid = sid * NC + lax.axis_index("c")
    base = wid * B_PER_W
    base_row = wid * R_PER_W

    @pl.when(sid == 0)
    def _stage_table():
        pltpu.sync_copy(table_hbm, table_sp)

    pltpu.sync_copy(ids_hbm.at[pl.ds(base, B_PER_W)], idx_v)
    plsc.subcore_barrier()

    gathers = []
    stores = []

    def gstart(i):
        gathers.append(
            pltpu.async_copy(
                table_sp.at[idx_v.at[pl.ds(i * CHUNK, CHUNK)]],
                bufs.at[i % NBUF],
                gsem,
            )
        )

    def sstart(i):
        stores.append(
            pltpu.async_copy(
                bufs.at[i % NBUF].reshape(CR, N_COLS, D),
                out_hbm.at[pl.ds(base_row + i * CR, CR)],
                ssem,
            )
        )

    for i in range(N_CHUNKS):
        if i >= NBUF:
            stores[i - NBUF].wait()
        gstart(i)
        if i >= LOOKAHEAD:
            j = i - LOOKAHEAD
            gathers[j].wait()
            sstart(j)
    for j in range(N_CHUNKS - LOOKAHEAD, N_CHUNKS):
        gathers[j].wait()
        sstart(j)
    for j in range(N_CHUNKS - NBUF, N_CHUNKS):
        stores[j].wait()


def kernel(camera_ids, camera_embeddings):
    ids = camera_ids.reshape(-1).astype(jnp.int32)
    return _embed(ids, camera_embeddings)


# speedup vs baseline: 17.0035x; 2.1938x over previous
import functools

import jax
import jax.numpy as jnp
from jax import lax
from jax.experimental import pallas as pl
from jax.experimental.pallas import tpu as pltpu
from jax.experimental.pallas import tpu_sc as plsc

N_ROWS = 16384
N_COLS = 6
N_COLS_PAD = 8
N_IDS = N_ROWS * N_COLS
D = 128

_info = plsc.get_sparse_core_info()
NC, NS = _info.num_cores, _info.num_subcores
NW = NC * NS
B_PER_W = N_IDS // NW
CHUNK = 128
N_CHUNKS = B_PER_W // CHUNK
NBUF = 4
LOOKAHEAD = 3

_mesh = plsc.VectorSubcoreMesh(core_axis_name="c", subcore_axis_name="s")


@functools.partial(
    pl.kernel,
    mesh=_mesh,
    out_type=jax.ShapeDtypeStruct((N_COLS, N_ROWS, D), jnp.float32),
    scratch_types=[
        pltpu.VMEM((B_PER_W,), jnp.int32),
        pltpu.VMEM((NBUF, CHUNK, D), jnp.float32),
        pltpu.VMEM_SHARED((8, D), jnp.float32),
        pltpu.SemaphoreType.DMA,
        pltpu.SemaphoreType.DMA,
    ],
)
def _embed(ids_hbm, table_hbm, out_hbm, idx_v, bufs, table_sp, gsem, ssem):
    sid = lax.axis_index("s")
    wid = sid * NC + lax.axis_index("c")
    base = wid * B_PER_W
    outf = out_hbm.reshape(N_IDS, D)

    @pl.when(sid == 0)
    def _stage_table():
        pltpu.sync_copy(table_hbm, table_sp)

    pltpu.sync_copy(ids_hbm.at[pl.ds(base, B_PER_W)], idx_v)
    plsc.subcore_barrier()

    gathers = []
    stores = []

    def gstart(i):
        gathers.append(
            pltpu.async_copy(
                table_sp.at[idx_v.at[pl.ds(i * CHUNK, CHUNK)]],
                bufs.at[i % NBUF],
                gsem,
            )
        )

    def sstart(i):
        stores.append(
            pltpu.async_copy(
                bufs.at[i % NBUF], outf.at[pl.ds(base + i * CHUNK, CHUNK)], ssem
            )
        )

    for i in range(N_CHUNKS):
        if i >= NBUF:
            stores[i - NBUF].wait()
        gstart(i)
        if i >= LOOKAHEAD:
            j = i - LOOKAHEAD
            gathers[j].wait()
            sstart(j)
    for j in range(N_CHUNKS - LOOKAHEAD, N_CHUNKS):
        gathers[j].wait()
        sstart(j)
    for j in range(N_CHUNKS - NBUF, N_CHUNKS):
        stores[j].wait()


def kernel(camera_ids, camera_embeddings):
    ids_t = jnp.pad(
        camera_ids.T.astype(jnp.int32), ((0, N_COLS_PAD - N_COLS), (0, 0))
    ).reshape(-1)
    out_t = _embed(ids_t, camera_embeddings)
    return jnp.swapaxes(out_t, 0, 1)


# speedup vs baseline: 17.0569x; 1.0031x over previous
import functools

import jax
import jax.numpy as jnp
from jax import lax
from jax.experimental import pallas as pl
from jax.experimental.pallas import tpu as pltpu
from jax.experimental.pallas import tpu_sc as plsc

N_ROWS = 16384
N_COLS = 6
N_COLS_PAD = 8
N_IDS = N_ROWS * N_COLS
D = 128

_info = plsc.get_sparse_core_info()
NC, NS = _info.num_cores, _info.num_subcores
NW = NC * NS
B_PER_W = N_IDS // NW
CHUNK = 128
N_CHUNKS = B_PER_W // CHUNK
NBUF = 6
LOOKAHEAD = 5

_mesh = plsc.VectorSubcoreMesh(core_axis_name="c", subcore_axis_name="s")


@functools.partial(
    pl.kernel,
    mesh=_mesh,
    out_type=jax.ShapeDtypeStruct((N_COLS, N_ROWS, D), jnp.float32),
    scratch_types=[
        pltpu.VMEM((B_PER_W,), jnp.int32),
        pltpu.VMEM((NBUF, CHUNK, D), jnp.float32),
        pltpu.VMEM_SHARED((8, D), jnp.float32),
        pltpu.SemaphoreType.DMA,
        pltpu.SemaphoreType.DMA,
    ],
)
def _embed(ids_hbm, table_hbm, out_hbm, idx_v, bufs, table_sp, gsem, ssem):
    sid = lax.axis_index("s")
    wid = sid * NC + lax.axis_index("c")
    base = wid * B_PER_W
    outf = out_hbm.reshape(N_IDS, D)

    @pl.when(sid == 0)
    def _stage_table():
        pltpu.sync_copy(table_hbm, table_sp)

    pltpu.sync_copy(ids_hbm.at[pl.ds(base, B_PER_W)], idx_v)
    plsc.subcore_barrier()

    gathers = []
    stores = []

    def gstart(i):
        gathers.append(
            pltpu.async_copy(
                table_sp.at[idx_v.at[pl.ds(i * CHUNK, CHUNK)]],
                bufs.at[i % NBUF],
                gsem,
            )
        )

    def sstart(i):
        stores.append(
            pltpu.async_copy(
                bufs.at[i % NBUF], outf.at[pl.ds(base + i * CHUNK, CHUNK)], ssem
            )
        )

    for i in range(N_CHUNKS):
        if i >= NBUF:
            stores[i - NBUF].wait()
        gstart(i)
        if i >= LOOKAHEAD:
            j = i - LOOKAHEAD
            gathers[j].wait()
            sstart(j)
    for j in range(N_CHUNKS - LOOKAHEAD, N_CHUNKS):
        gathers[j].wait()
        sstart(j)
    for j in range(N_CHUNKS - NBUF, N_CHUNKS):
        stores[j].wait()


def kernel(camera_ids, camera_embeddings):
    ids_t = jnp.pad(
        camera_ids.T.astype(jnp.int32), ((0, N_COLS_PAD - N_COLS), (0, 0))
    ).reshape(-1)
    out_t = _embed(ids_t, camera_embeddings)
    return jnp.swapaxes(out_t, 0, 1)
